# Initial kernel scaffold; baseline (speedup 1.0000x reference)
import jax
import jax.numpy as jnp
from jax.experimental import pallas as pl


def kernel(A, X, W, b):
    raise NotImplementedError("write your pallas kernel here")


# speedup vs baseline: 1.2242x; 1.2242x over previous
import jax
import jax.numpy as jnp
from jax import lax
from jax.experimental import pallas as pl
from jax.experimental.pallas import tpu as pltpu

_LANE = 128


def _sr_kernel(a_ref, x_ref, wt_ref, b_ref, o_ref):
    i = pl.program_id(0)
    tm = o_ref.shape[0]

    a = a_ref[...]
    d = jnp.sum(a, axis=0, keepdims=True) + 1.0

    sub = a_ref[pl.ds(i * tm, tm), :]
    rows = lax.broadcasted_iota(jnp.int32, (tm, tm), 0)
    cols = lax.broadcasted_iota(jnp.int32, (tm, tm), 1)
    diag = jnp.sum(jnp.where(rows == cols, sub, 0.0),
                   axis=0, keepdims=True)

    scale = (diag + 1.0) / d
    xs = x_ref[...] * scale

    out = lax.dot_general(
        xs, wt_ref[...],
        dimension_numbers=(((0,), (0,)), ((), ())),
        preferred_element_type=jnp.float32)
    o_ref[...] = jnp.maximum(out + b_ref[...], 0.0)


def kernel(A, X, W, b, *, block_n=None):
    n = A.shape[0]
    in_units = X.shape[0]
    out_units = W.shape[0]
    out_pad = pl.cdiv(out_units, _LANE) * _LANE

    if block_n is not None:
        tm = block_n
    elif n <= 512:
        tm = n
    else:
        tm = 256
    n_pad = pl.cdiv(n, tm) * tm

    A = A.astype(jnp.float32)
    X = X.astype(jnp.float32)
    if n_pad != n:
        pad = n_pad - n
        A = jnp.pad(A, ((0, pad), (0, pad)))
        X = jnp.pad(X, ((0, 0), (0, pad)))

    wt = jnp.pad(jnp.transpose(W).astype(jnp.float32),
                 ((0, 0), (0, out_pad - out_units)))
    b2 = jnp.pad(b.astype(jnp.float32).reshape(1, out_units),
                 ((0, 0), (0, out_pad - out_units)))

    out = pl.pallas_call(
        _sr_kernel,
        out_shape=jax.ShapeDtypeStruct((n_pad, out_pad), jnp.float32),
        grid=(n_pad // tm,),
        in_specs=[
            pl.BlockSpec((n_pad, tm), lambda i: (0, i)),
            pl.BlockSpec((in_units, tm), lambda i: (0, i)),
            pl.BlockSpec((in_units, out_pad), lambda i: (0, 0)),
            pl.BlockSpec((1, out_pad), lambda i: (0, 0)),
        ],
        out_specs=pl.BlockSpec((tm, out_pad), lambda i: (i, 0)),
        compiler_params=pltpu.CompilerParams(
            dimension_semantics=("parallel",)),
    )(A, X, wt, b2)

    return out[:n, :out_units]


# speedup vs baseline: 1.2371x; 1.0105x over previous
import jax
import jax.numpy as jnp
from jax import lax
from jax.experimental import pallas as pl
from jax.experimental.pallas import tpu as pltpu

_LANE = 128


def _sr_kernel(a_ref, x_ref, wt_ref, b_ref, o_ref):
    i = pl.program_id(0)
    tm = o_ref.shape[0]

    a = a_ref[...]
    d = jnp.sum(a, axis=0, keepdims=True) + 1.0

    sub = a_ref[pl.ds(i * tm, tm), :]
    rows = lax.broadcasted_iota(jnp.int32, (tm, tm), 0)
    cols = lax.broadcasted_iota(jnp.int32, (tm, tm), 1)
    diag = jnp.sum(jnp.where(rows == cols, sub, 0.0),
                   axis=0, keepdims=True)

    scale = (diag + 1.0) / d
    xs = x_ref[...] * scale

    out = lax.dot_general(
        xs, wt_ref[...],
        dimension_numbers=(((0,), (0,)), ((), ())),
        preferred_element_type=jnp.float32)
    o_ref[...] = jnp.maximum(out + b_ref[...], 0.0)


def kernel(A, X, W, b, *, block_n=None):
    n = A.shape[0]
    in_units = X.shape[0]
    out_units = W.shape[0]
    out_pad = pl.cdiv(out_units, _LANE) * _LANE

    if block_n is not None:
        tm = block_n
    elif n <= 512:
        tm = n
    else:
        tm = 512
    n_pad = pl.cdiv(n, tm) * tm

    A = A.astype(jnp.float32)
    X = X.astype(jnp.float32)
    if n_pad != n:
        pad = n_pad - n
        A = jnp.pad(A, ((0, pad), (0, pad)))
        X = jnp.pad(X, ((0, 0), (0, pad)))

    wt = jnp.pad(jnp.transpose(W).astype(jnp.float32),
                 ((0, 0), (0, out_pad - out_units)))
    b2 = jnp.pad(b.astype(jnp.float32).reshape(1, out_units),
                 ((0, 0), (0, out_pad - out_units)))

    out = pl.pallas_call(
        _sr_kernel,
        out_shape=jax.ShapeDtypeStruct((n_pad, out_pad), jnp.float32),
        grid=(n_pad // tm,),
        in_specs=[
            pl.BlockSpec((n_pad, tm), lambda i: (0, i)),
            pl.BlockSpec((in_units, tm), lambda i: (0, i)),
            pl.BlockSpec((in_units, out_pad), lambda i: (0, 0)),
            pl.BlockSpec((1, out_pad), lambda i: (0, 0)),
        ],
        out_specs=pl.BlockSpec((tm, out_pad), lambda i: (i, 0)),
        compiler_params=pltpu.CompilerParams(
            dimension_semantics=("parallel",)),
    )(A, X, wt, b2)

    return out[:n, :out_units]


# speedup vs baseline: 1.2589x; 1.0176x over previous
import jax
import jax.numpy as jnp
from jax import lax
from jax.experimental import pallas as pl
from jax.experimental.pallas import tpu as pltpu

_LANE = 128


def _sr_kernel(a_ref, x_ref, w_ref, b_ref, o_ref):
    i = pl.program_id(0)
    tm = o_ref.shape[0]

    a = a_ref[...]
    d = jnp.sum(a, axis=0, keepdims=True) + 1.0

    sub = a_ref[pl.ds(i * tm, tm), :]
    rows = lax.broadcasted_iota(jnp.int32, (tm, tm), 0)
    cols = lax.broadcasted_iota(jnp.int32, (tm, tm), 1)
    diag = jnp.sum(jnp.where(rows == cols, sub, 0.0),
                   axis=0, keepdims=True)

    scale = (diag + 1.0) / d
    xs = x_ref[...] * scale

    out = lax.dot_general(
        xs, w_ref[...],
        dimension_numbers=(((0,), (1,)), ((), ())),
        preferred_element_type=jnp.float32)
    o_ref[...] = jnp.maximum(out + b_ref[...], 0.0)


def kernel(A, X, W, b, *, block_n=None):
    n = A.shape[0]
    in_units = X.shape[0]
    out_units = W.shape[0]
    out_pad = pl.cdiv(out_units, _LANE) * _LANE

    if block_n is not None:
        tm = block_n
    elif n <= 512:
        tm = n
    else:
        tm = 512
    n_pad = pl.cdiv(n, tm) * tm

    A = A.astype(jnp.float32)
    X = X.astype(jnp.float32)
    if n_pad != n:
        pad = n_pad - n
        A = jnp.pad(A, ((0, pad), (0, pad)))
        X = jnp.pad(X, ((0, 0), (0, pad)))

    w2 = jnp.pad(W.astype(jnp.float32),
                 ((0, out_pad - out_units), (0, 0)))
    b2 = jnp.pad(b.astype(jnp.float32).reshape(1, out_units),
                 ((0, 0), (0, out_pad - out_units)))

    out = pl.pallas_call(
        _sr_kernel,
        out_shape=jax.ShapeDtypeStruct((n_pad, out_pad), jnp.float32),
        grid=(n_pad // tm,),
        in_specs=[
            pl.BlockSpec((n_pad, tm), lambda i: (0, i)),
            pl.BlockSpec((in_units, tm), lambda i: (0, i)),
            pl.BlockSpec((out_pad, in_units), lambda i: (0, 0)),
            pl.BlockSpec((1, out_pad), lambda i: (0, 0)),
        ],
        out_specs=pl.BlockSpec((tm, out_pad), lambda i: (i, 0)),
        compiler_params=pltpu.CompilerParams(
            dimension_semantics=("parallel",)),
    )(A, X, w2, b2)

    return out[:n, :out_units]
